# baseline (speedup 1.0000x reference)
import functools

import jax
import jax.numpy as jnp
from jax import lax
from jax.experimental import pallas as pl
from jax.experimental.pallas import tpu as pltpu
from jax.experimental.pallas import tpu_sc as plsc

NC, NS, L = 2, 16, 16
NW = NC * NS
C = 64


def _dense1_body(x_ref, w1t_ref, asrc_map_ref, adst_map_ref, hx_ref, adst_ref):
    h = jnp.dot(x_ref[...], w1t_ref[...], preferred_element_type=jnp.float32)
    asrc = jnp.dot(h, asrc_map_ref[...], precision=lax.Precision.HIGHEST)
    zpad = jnp.zeros((h.shape[0], 8), jnp.float32)
    hx_ref[...] = jnp.concatenate([h, asrc, zpad], axis=1)
    adst_ref[...] = jnp.dot(h, adst_map_ref[...], precision=lax.Precision.HIGHEST)


def _combine1_body(acc0_ref, acc1_ref, hx_ref, adst_ref, b1_ref, w2t_ref,
                   a2s_map_ref, a2d_map_ref, bc8_ref, hx2_ref, adst2_ref):
    acc0 = acc0_ref[0]
    acc1 = acc1_ref[0]
    asrc = hx_ref[:, 128:136]
    ad = adst_ref[:, 0:8]
    a = asrc + ad
    a = jnp.where(a > 0, a, 0.2 * a)
    exs = jnp.exp(a)
    den = acc0[:, 128:136] + acc1[:, 128:136] + exs
    h = hx_ref[:, 0:128]
    bc8 = bc8_ref[...]
    exs_b = jnp.dot(exs, bc8, precision=lax.Precision.HIGHEST)
    num = acc0[:, 0:128] + acc1[:, 0:128] + exs_b * h
    recip = 1.0 / (den + 1e-16)
    recip_b = jnp.dot(recip, bc8, precision=lax.Precision.HIGHEST)
    out1 = num * recip_b + b1_ref[...]
    g = jnp.where(out1 > 0, out1, jnp.exp(out1) - 1.0)
    h2 = jnp.dot(g, w2t_ref[...], preferred_element_type=jnp.float32)
    asrc2 = jnp.dot(h2, a2s_map_ref[...], precision=lax.Precision.HIGHEST)
    zpad = jnp.zeros((h2.shape[0], 8), jnp.float32)
    hx2_ref[...] = jnp.concatenate([h2, asrc2, zpad], axis=1)
    adst2_ref[...] = jnp.dot(h2, a2d_map_ref[...], precision=lax.Precision.HIGHEST)


def _combine2_body(acc0_ref, acc1_ref, hx2_ref, adst2_ref, b2_ref, p8_ref,
                   out_ref):
    acc0 = acc0_ref[0]
    acc1 = acc1_ref[0]
    asrc = hx2_ref[:, 64:72]
    ad = adst2_ref[:, 0:8]
    a = asrc + ad
    a = jnp.where(a > 0, a, 0.2 * a)
    exs = jnp.exp(a)
    den = acc0[:, 64:72] + acc1[:, 64:72] + exs
    h2 = hx2_ref[:, 0:64]
    p8 = p8_ref[...]
    exs_b = jnp.dot(exs, p8, precision=lax.Precision.HIGHEST)
    num = acc0[:, 0:64] + acc1[:, 0:64] + exs_b * h2
    recip = 1.0 / (den + 1e-16)
    recip_b = jnp.dot(recip, p8, precision=lax.Precision.HIGHEST)
    out_ref[...] = num * recip_b + b2_ref[...]


def _make_edge_kernel(D, NH, ACC_N, CH):
    W = D + 16
    HB = D // NH
    RPT = ACC_N // NS
    mesh = plsc.VectorSubcoreMesh(core_axis_name="c", subcore_axis_name="s")

    assert CH % 2 == 0

    @functools.partial(
        pl.kernel,
        out_type=jax.ShapeDtypeStruct((NC, ACC_N, W), jnp.float32),
        mesh=mesh,
        compiler_params=pltpu.CompilerParams(use_tc_tiling_on_sc=False,
                                             needs_layout_passes=False),
        scratch_types=[
            [pltpu.VMEM((C,), jnp.int32)] * 2,
            [pltpu.VMEM((C,), jnp.int32)] * 2,
            [pltpu.VMEM((C,), jnp.int32)] * 2,
            [pltpu.VMEM((C, W), jnp.float32)] * 2,
            [pltpu.VMEM((C, 16), jnp.float32)] * 2,
            [pltpu.VMEM((C, W), jnp.float32)] * 2,
            pltpu.VMEM_SHARED((ACC_N, W), jnp.float32),
            [pltpu.SemaphoreType.DMA] * 2,
            [pltpu.SemaphoreType.DMA] * 2,
        ],
    )
    def edge_kernel(hx_hbm, adst_hbm, src_hbm, dstg_hbm, dsts_hbm, out_hbm,
                    srcv, dstv_g, dstv_s, hxv, adstv, msgv, acc, gsem, ssem):
        cid = lax.axis_index("c")
        sid = lax.axis_index("s")
        wid = sid * NC + cid

        def _zero_row(r, carry):
            for c0 in range(W // 16):
                msgv[0][r, pl.ds(c0 * 16, 16)] = jnp.zeros((16,), jnp.float32)
            return carry
        lax.fori_loop(0, C, _zero_row, 0)
        row0 = sid * RPT
        off = 0
        while off < RPT:
            nb = min(C, RPT - off)
            pltpu.sync_copy(msgv[0].at[pl.ds(0, nb)],
                            acc.at[pl.ds(row0 + off, nb)])
            off += nb
        plsc.subcore_barrier()

        ebase = wid * (CH * C)

        def _fire_gather(b, ci):
            base = ebase + ci * C
            pltpu.sync_copy(src_hbm.at[pl.ds(base, C)], srcv[b])
            pltpu.sync_copy(dstg_hbm.at[pl.ds(base, C)], dstv_g[b])
            pltpu.async_copy(hx_hbm.at[srcv[b]], hxv[b], gsem[b])
            pltpu.async_copy(adst_hbm.at[dstv_g[b]], adstv[b], gsem[b])

        def _turn(b, ci):
            pltpu.make_async_copy(hx_hbm.at[srcv[b]], hxv[b], gsem[b]).wait()
            pltpu.make_async_copy(adst_hbm.at[dstv_g[b]], adstv[b],
                                  gsem[b]).wait()
            @pl.when(ci >= 2)
            def _():
                pltpu.make_async_copy(msgv[b], acc.at[dstv_s[b]],
                                      ssem[b]).wait()
            pltpu.sync_copy(dsts_hbm.at[pl.ds(ebase + ci * C, C)], dstv_s[b])

            def _ex4(i, ecarry):
                for j in range(4):
                    e = i * 4 + j
                    asrc = hxv[b][e, pl.ds(D, 16)]
                    ad = adstv[b][e, :]
                    a = asrc + ad
                    a = jnp.where(a > 0, a, 0.2 * a)
                    ex = jnp.exp(a)
                    adstv[b][e, :] = ex
                    msgv[b][e, pl.ds(D, 16)] = ex
                return ecarry
            lax.fori_loop(0, C // 4, _ex4, 0)

            def _msg4(i, ecarry):
                for j in range(4):
                    e = i * 4 + j
                    eidx = jnp.broadcast_to(e, (L,)).astype(jnp.int32)
                    for hd in range(NH):
                        cidx = jnp.full((L,), hd, jnp.int32)
                        bb = plsc.load_gather(adstv[b], [eidx, cidx])
                        for v in range(HB // 16):
                            c0 = hd * HB + v * 16
                            msgv[b][e, pl.ds(c0, 16)] = (
                                hxv[b][e, pl.ds(c0, 16)] * bb)
                return ecarry
            lax.fori_loop(0, C // 4, _msg4, 0)
            pltpu.async_copy(msgv[b], acc.at[dstv_s[b]], ssem[b], add=True)

            @pl.when(ci + 2 < CH)
            def _():
                _fire_gather(b, ci + 2)

        _fire_gather(0, 0)
        _fire_gather(1, 1)

        def _pair(k, carry):
            _turn(0, 2 * k)
            _turn(1, 2 * k + 1)
            return carry
        lax.fori_loop(0, CH // 2, _pair, 0)
        for b in range(2):
            pltpu.make_async_copy(msgv[b], acc.at[dstv_s[b]], ssem[b]).wait()
        plsc.subcore_barrier()

        off = 0
        while off < RPT:
            nb = min(C, RPT - off)
            pltpu.sync_copy(acc.at[pl.ds(row0 + off, nb)],
                            msgv[0].at[pl.ds(0, nb)])
            pltpu.sync_copy(msgv[0].at[pl.ds(0, nb)],
                            out_hbm.at[cid, pl.ds(row0 + off, nb)])
            off += nb

    return edge_kernel


def kernel(x, edge_index, W1, att_src1, att_dst1, b1, W2, att_src2, att_dst2,
           b2):
    N, d_in = x.shape
    E = edge_index.shape[1]
    heads, hf = att_src1.shape[1], att_src1.shape[2]
    D1 = heads * hf
    n_cls = W2.shape[0]
    ACC_N = 10112
    f32 = jnp.float32

    EPC = NW * C
    CH = -(-E // EPC)
    CH += CH % 2
    E_pad = CH * EPC
    pad = E_pad - E
    src_p = jnp.concatenate([edge_index[0], jnp.zeros((pad,), jnp.int32)])
    dstg_p = jnp.concatenate([edge_index[1], jnp.zeros((pad,), jnp.int32)])
    dsts_p = jnp.concatenate([edge_index[1], jnp.full((pad,), N, jnp.int32)])

    att1s = att_src1.reshape(D1)
    att1d = att_dst1.reshape(D1)
    headsel = (jnp.arange(D1)[:, None] // hf ==
               jnp.arange(heads)[None, :]).astype(f32)
    asrc_map = headsel * att1s[:, None]
    adst_map = jnp.pad(headsel * att1d[:, None], ((0, 0), (0, 8)))
    bc8 = headsel.T
    a2s_map = jnp.pad(att_src2.reshape(n_cls, 1), ((0, 0), (0, 7)))
    a2d_map = jnp.pad(att_dst2.reshape(n_cls, 1), ((0, 0), (0, 15)))
    p8 = jnp.zeros((8, n_cls), f32).at[0, :].set(1.0)
    b1r = b1.reshape(1, D1)
    b2r = b2.reshape(1, n_cls)

    BN = 1000
    hx, adst16 = pl.pallas_call(
        _dense1_body,
        grid=(N // BN,),
        in_specs=[
            pl.BlockSpec((BN, d_in), lambda i: (i, 0)),
            pl.BlockSpec((d_in, D1), lambda i: (0, 0)),
            pl.BlockSpec((D1, heads), lambda i: (0, 0)),
            pl.BlockSpec((D1, 16), lambda i: (0, 0)),
        ],
        out_specs=[
            pl.BlockSpec((BN, D1 + 16), lambda i: (i, 0)),
            pl.BlockSpec((BN, 16), lambda i: (i, 0)),
        ],
        out_shape=[
            jax.ShapeDtypeStruct((N, D1 + 16), f32),
            jax.ShapeDtypeStruct((N, 16), f32),
        ],
    )(x, W1.T, asrc_map, adst_map)

    edge1 = _make_edge_kernel(D1, heads, ACC_N, CH)
    acc1 = edge1(hx, adst16, src_p, dstg_p, dsts_p)

    hx2, adst2 = pl.pallas_call(
        _combine1_body,
        grid=(N // BN,),
        in_specs=[
            pl.BlockSpec((1, BN, D1 + 16), lambda i: (0, i, 0)),
            pl.BlockSpec((1, BN, D1 + 16), lambda i: (1, i, 0)),
            pl.BlockSpec((BN, D1 + 16), lambda i: (i, 0)),
            pl.BlockSpec((BN, 16), lambda i: (i, 0)),
            pl.BlockSpec((1, D1), lambda i: (0, 0)),
            pl.BlockSpec((D1, n_cls), lambda i: (0, 0)),
            pl.BlockSpec((n_cls, 8), lambda i: (0, 0)),
            pl.BlockSpec((n_cls, 16), lambda i: (0, 0)),
            pl.BlockSpec((8, D1), lambda i: (0, 0)),
        ],
        out_specs=[
            pl.BlockSpec((BN, n_cls + 16), lambda i: (i, 0)),
            pl.BlockSpec((BN, 16), lambda i: (i, 0)),
        ],
        out_shape=[
            jax.ShapeDtypeStruct((N, n_cls + 16), f32),
            jax.ShapeDtypeStruct((N, 16), f32),
        ],
    )(acc1, acc1, hx, adst16, b1r, W2.T, a2s_map, a2d_map, bc8)

    edge2 = _make_edge_kernel(n_cls, 1, ACC_N, CH)
    acc2 = edge2(hx2, adst2, src_p, dstg_p, dsts_p)

    out = pl.pallas_call(
        _combine2_body,
        grid=(N // BN,),
        in_specs=[
            pl.BlockSpec((1, BN, n_cls + 16), lambda i: (0, i, 0)),
            pl.BlockSpec((1, BN, n_cls + 16), lambda i: (1, i, 0)),
            pl.BlockSpec((BN, n_cls + 16), lambda i: (i, 0)),
            pl.BlockSpec((BN, 16), lambda i: (i, 0)),
            pl.BlockSpec((1, n_cls), lambda i: (0, 0)),
            pl.BlockSpec((8, n_cls), lambda i: (0, 0)),
        ],
        out_specs=pl.BlockSpec((BN, n_cls), lambda i: (i, 0)),
        out_shape=jax.ShapeDtypeStruct((N, n_cls), f32),
    )(acc2, acc2, hx2, adst2, b2r, p8)

    return out


# speedup vs baseline: 1.2040x; 1.2040x over previous
import functools

import jax
import jax.numpy as jnp
from jax import lax
from jax.experimental import pallas as pl
from jax.experimental.pallas import tpu as pltpu
from jax.experimental.pallas import tpu_sc as plsc

NC, NS, L = 2, 16, 16
NW = NC * NS


def _dense1_body(x_ref, w1t_ref, asrc_map_ref, adst_map_ref, hx_ref, adst_ref):
    h = jnp.dot(x_ref[...], w1t_ref[...], preferred_element_type=jnp.float32)
    asrc = jnp.dot(h, asrc_map_ref[...], precision=lax.Precision.HIGHEST)
    zpad = jnp.zeros((h.shape[0], 8), jnp.float32)
    hx_ref[...] = jnp.concatenate([h, asrc, zpad], axis=1)
    adst_ref[...] = jnp.dot(h, adst_map_ref[...], precision=lax.Precision.HIGHEST)


def _combine1_body(acc0_ref, acc1_ref, hx_ref, adst_ref, b1_ref, w2t_ref,
                   a2s_map_ref, a2d_map_ref, bc8_ref, hx2_ref, adst2_ref):
    acc0 = acc0_ref[0]
    acc1 = acc1_ref[0]
    asrc = hx_ref[:, 128:136]
    ad = adst_ref[:, 0:8]
    a = asrc + ad
    a = jnp.where(a > 0, a, 0.2 * a)
    exs = jnp.exp(a)
    den = acc0[:, 128:136] + acc1[:, 128:136] + exs
    h = hx_ref[:, 0:128]
    bc8 = bc8_ref[...]
    exs_b = jnp.dot(exs, bc8, precision=lax.Precision.HIGHEST)
    num = acc0[:, 0:128] + acc1[:, 0:128] + exs_b * h
    recip = 1.0 / (den + 1e-16)
    recip_b = jnp.dot(recip, bc8, precision=lax.Precision.HIGHEST)
    out1 = num * recip_b + b1_ref[...]
    g = jnp.where(out1 > 0, out1, jnp.exp(out1) - 1.0)
    h2 = jnp.dot(g, w2t_ref[...], preferred_element_type=jnp.float32)
    asrc2 = jnp.dot(h2, a2s_map_ref[...], precision=lax.Precision.HIGHEST)
    zpad = jnp.zeros((h2.shape[0], 8), jnp.float32)
    hx2_ref[...] = jnp.concatenate([h2, asrc2, zpad], axis=1)
    adst2_ref[...] = jnp.dot(h2, a2d_map_ref[...], precision=lax.Precision.HIGHEST)


def _combine2_body(acc0_ref, acc1_ref, hx2_ref, adst2_ref, b2_ref, p8_ref,
                   out_ref):
    acc0 = acc0_ref[0]
    acc1 = acc1_ref[0]
    asrc = hx2_ref[:, 64:72]
    ad = adst2_ref[:, 0:8]
    a = asrc + ad
    a = jnp.where(a > 0, a, 0.2 * a)
    exs = jnp.exp(a)
    den = acc0[:, 64:72] + acc1[:, 64:72] + exs
    h2 = hx2_ref[:, 0:64]
    p8 = p8_ref[...]
    exs_b = jnp.dot(exs, p8, precision=lax.Precision.HIGHEST)
    num = acc0[:, 0:64] + acc1[:, 0:64] + exs_b * h2
    recip = 1.0 / (den + 1e-16)
    recip_b = jnp.dot(recip, p8, precision=lax.Precision.HIGHEST)
    out_ref[...] = num * recip_b + b2_ref[...]


def _make_edge_kernel(D, NH, ACC_N, CH, C):
    W = D + 16
    HB = D // NH
    RPT = ACC_N // NS
    mesh = plsc.VectorSubcoreMesh(core_axis_name="c", subcore_axis_name="s")

    assert CH % 4 == 0 and CH >= 8

    @functools.partial(
        pl.kernel,
        out_type=jax.ShapeDtypeStruct((NC, ACC_N, W), jnp.float32),
        mesh=mesh,
        compiler_params=pltpu.CompilerParams(use_tc_tiling_on_sc=False,
                                             needs_layout_passes=False),
        scratch_types=[
            [pltpu.VMEM((C,), jnp.int32)] * 4,
            [pltpu.VMEM((C,), jnp.int32)] * 4,
            [pltpu.VMEM((C,), jnp.int32)] * 4,
            [pltpu.VMEM((C, W), jnp.float32)] * 2,
            [pltpu.VMEM((C, 16), jnp.float32)] * 2,
            [pltpu.VMEM((C, W), jnp.float32)] * 2,
            pltpu.VMEM_SHARED((ACC_N, W), jnp.float32),
            [pltpu.SemaphoreType.DMA] * 2,
            [pltpu.SemaphoreType.DMA] * 2,
            [pltpu.SemaphoreType.DMA] * 4,
            [pltpu.SemaphoreType.DMA] * 4,
        ],
    )
    def edge_kernel(hx_hbm, adst_hbm, src_hbm, dstg_hbm, dsts_hbm, out_hbm,
                    srcv, dstgv, dstsv, hxv, adstv, msgv, acc, gsem, ssem,
                    isem, dsem):
        cid = lax.axis_index("c")
        sid = lax.axis_index("s")
        wid = sid * NC + cid

        def _zero_row(r, carry):
            for c0 in range(W // 16):
                msgv[0][r, pl.ds(c0 * 16, 16)] = jnp.zeros((16,), jnp.float32)
            return carry
        lax.fori_loop(0, C, _zero_row, 0)
        row0 = sid * RPT
        off = 0
        while off < RPT:
            nb = min(C, RPT - off)
            pltpu.sync_copy(msgv[0].at[pl.ds(0, nb)],
                            acc.at[pl.ds(row0 + off, nb)])
            off += nb
        plsc.subcore_barrier()

        ebase = wid * (CH * C)

        def _turn(b, s, ci):
            s2 = (s + 2) % 4

            pltpu.make_async_copy(hx_hbm.at[srcv[s]], hxv[b], gsem[b]).wait()
            pltpu.make_async_copy(adst_hbm.at[dstgv[s]], adstv[b],
                                  gsem[b]).wait()
            @pl.when(ci + 4 < CH)
            def _():
                base4 = ebase + (ci + 4) * C
                pltpu.async_copy(src_hbm.at[pl.ds(base4, C)], srcv[s],
                                 isem[s])
                pltpu.async_copy(dstg_hbm.at[pl.ds(base4, C)], dstgv[s],
                                 isem[s])

            @pl.when(ci >= 2)
            def _():
                pltpu.make_async_copy(msgv[b], acc.at[dstsv[s2]],
                                      ssem[b]).wait()
            @pl.when(ci + 2 < CH)
            def _():
                pltpu.async_copy(dsts_hbm.at[pl.ds(ebase + (ci + 2) * C, C)],
                                 dstsv[s2], dsem[s2])

            def _ex4(i, ecarry):
                for j in range(4):
                    e = i * 4 + j
                    asrc = hxv[b][e, pl.ds(D, 16)]
                    ad = adstv[b][e, :]
                    a = asrc + ad
                    a = jnp.where(a > 0, a, 0.2 * a)
                    ex = jnp.exp(a)
                    adstv[b][e, :] = ex
                    msgv[b][e, pl.ds(D, 16)] = ex
                return ecarry
            lax.fori_loop(0, C // 4, _ex4, 0)

            def _msg4(i, ecarry):
                for j in range(4):
                    e = i * 4 + j
                    eidx = jnp.broadcast_to(e, (L,)).astype(jnp.int32)
                    for hd in range(NH):
                        cidx = jnp.full((L,), hd, jnp.int32)
                        bb = plsc.load_gather(adstv[b], [eidx, cidx])
                        for v in range(HB // 16):
                            c0 = hd * HB + v * 16
                            msgv[b][e, pl.ds(c0, 16)] = (
                                hxv[b][e, pl.ds(c0, 16)] * bb)
                return ecarry
            lax.fori_loop(0, C // 4, _msg4, 0)

            @pl.when(ci >= 2)
            def _():
                pltpu.make_async_copy(dsts_hbm.at[pl.ds(ebase + ci * C, C)],
                                      dstsv[s], dsem[s]).wait()
            pltpu.async_copy(msgv[b], acc.at[dstsv[s]], ssem[b], add=True)

            @pl.when(ci + 2 < CH)
            def _():
                @pl.when(ci >= 2)
                def _():
                    base2 = ebase + (ci + 2) * C
                    pltpu.make_async_copy(src_hbm.at[pl.ds(base2, C)],
                                          srcv[s2], isem[s2]).wait()
                    pltpu.make_async_copy(dstg_hbm.at[pl.ds(base2, C)],
                                          dstgv[s2], isem[s2]).wait()
                pltpu.async_copy(hx_hbm.at[srcv[s2]], hxv[b], gsem[b])
                pltpu.async_copy(adst_hbm.at[dstgv[s2]], adstv[b], gsem[b])

        for s in range(4):
            pltpu.sync_copy(src_hbm.at[pl.ds(ebase + s * C, C)], srcv[s])
            pltpu.sync_copy(dstg_hbm.at[pl.ds(ebase + s * C, C)], dstgv[s])
        for s in range(2):
            pltpu.sync_copy(dsts_hbm.at[pl.ds(ebase + s * C, C)], dstsv[s])
            pltpu.async_copy(hx_hbm.at[srcv[s]], hxv[s], gsem[s])
            pltpu.async_copy(adst_hbm.at[dstgv[s]], adstv[s], gsem[s])

        def _quad(k, carry):
            ci = 4 * k
            _turn(0, 0, ci)
            _turn(1, 1, ci + 1)
            _turn(0, 2, ci + 2)
            _turn(1, 3, ci + 3)
            return carry
        lax.fori_loop(0, CH // 4, _quad, 0)
        for b, s in ((0, (CH - 2) % 4), (1, (CH - 1) % 4)):
            pltpu.make_async_copy(msgv[b], acc.at[dstsv[s]], ssem[b]).wait()
        plsc.subcore_barrier()

        off = 0
        while off < RPT:
            nb = min(C, RPT - off)
            pltpu.sync_copy(acc.at[pl.ds(row0 + off, nb)],
                            msgv[0].at[pl.ds(0, nb)])
            pltpu.sync_copy(msgv[0].at[pl.ds(0, nb)],
                            out_hbm.at[cid, pl.ds(row0 + off, nb)])
            off += nb

    return edge_kernel


def kernel(x, edge_index, W1, att_src1, att_dst1, b1, W2, att_src2, att_dst2,
           b2):
    N, d_in = x.shape
    E = edge_index.shape[1]
    heads, hf = att_src1.shape[1], att_src1.shape[2]
    D1 = heads * hf
    n_cls = W2.shape[0]
    ACC_N = 10112
    f32 = jnp.float32

    def _pad_edges(C):
        EPC = NW * C
        CH = -(-E // EPC)
        CH = -(-CH // 4) * 4
        pad = CH * EPC - E
        src_p = jnp.concatenate([edge_index[0], jnp.zeros((pad,), jnp.int32)])
        dstg_p = jnp.concatenate([edge_index[1], jnp.zeros((pad,), jnp.int32)])
        dsts_p = jnp.concatenate([edge_index[1],
                                  jnp.full((pad,), N, jnp.int32)])
        return CH, src_p, dstg_p, dsts_p

    C1, C2 = 64, 128
    CH1, src_p1, dstg_p1, dsts_p1 = _pad_edges(C1)
    CH2, src_p2, dstg_p2, dsts_p2 = _pad_edges(C2)

    att1s = att_src1.reshape(D1)
    att1d = att_dst1.reshape(D1)
    headsel = (jnp.arange(D1)[:, None] // hf ==
               jnp.arange(heads)[None, :]).astype(f32)
    asrc_map = headsel * att1s[:, None]
    adst_map = jnp.pad(headsel * att1d[:, None], ((0, 0), (0, 8)))
    bc8 = headsel.T
    a2s_map = jnp.pad(att_src2.reshape(n_cls, 1), ((0, 0), (0, 7)))
    a2d_map = jnp.pad(att_dst2.reshape(n_cls, 1), ((0, 0), (0, 15)))
    p8 = jnp.zeros((8, n_cls), f32).at[0, :].set(1.0)
    b1r = b1.reshape(1, D1)
    b2r = b2.reshape(1, n_cls)

    BN = 1000
    hx, adst16 = pl.pallas_call(
        _dense1_body,
        grid=(N // BN,),
        in_specs=[
            pl.BlockSpec((BN, d_in), lambda i: (i, 0)),
            pl.BlockSpec((d_in, D1), lambda i: (0, 0)),
            pl.BlockSpec((D1, heads), lambda i: (0, 0)),
            pl.BlockSpec((D1, 16), lambda i: (0, 0)),
        ],
        out_specs=[
            pl.BlockSpec((BN, D1 + 16), lambda i: (i, 0)),
            pl.BlockSpec((BN, 16), lambda i: (i, 0)),
        ],
        out_shape=[
            jax.ShapeDtypeStruct((N, D1 + 16), f32),
            jax.ShapeDtypeStruct((N, 16), f32),
        ],
    )(x, W1.T, asrc_map, adst_map)

    edge1 = _make_edge_kernel(D1, heads, ACC_N, CH1, C1)
    acc1 = edge1(hx, adst16, src_p1, dstg_p1, dsts_p1)

    hx2, adst2 = pl.pallas_call(
        _combine1_body,
        grid=(N // BN,),
        in_specs=[
            pl.BlockSpec((1, BN, D1 + 16), lambda i: (0, i, 0)),
            pl.BlockSpec((1, BN, D1 + 16), lambda i: (1, i, 0)),
            pl.BlockSpec((BN, D1 + 16), lambda i: (i, 0)),
            pl.BlockSpec((BN, 16), lambda i: (i, 0)),
            pl.BlockSpec((1, D1), lambda i: (0, 0)),
            pl.BlockSpec((D1, n_cls), lambda i: (0, 0)),
            pl.BlockSpec((n_cls, 8), lambda i: (0, 0)),
            pl.BlockSpec((n_cls, 16), lambda i: (0, 0)),
            pl.BlockSpec((8, D1), lambda i: (0, 0)),
        ],
        out_specs=[
            pl.BlockSpec((BN, n_cls + 16), lambda i: (i, 0)),
            pl.BlockSpec((BN, 16), lambda i: (i, 0)),
        ],
        out_shape=[
            jax.ShapeDtypeStruct((N, n_cls + 16), f32),
            jax.ShapeDtypeStruct((N, 16), f32),
        ],
    )(acc1, acc1, hx, adst16, b1r, W2.T, a2s_map, a2d_map, bc8)

    edge2 = _make_edge_kernel(n_cls, 1, ACC_N, CH2, C2)
    acc2 = edge2(hx2, adst2, src_p2, dstg_p2, dsts_p2)

    out = pl.pallas_call(
        _combine2_body,
        grid=(N // BN,),
        in_specs=[
            pl.BlockSpec((1, BN, n_cls + 16), lambda i: (0, i, 0)),
            pl.BlockSpec((1, BN, n_cls + 16), lambda i: (1, i, 0)),
            pl.BlockSpec((BN, n_cls + 16), lambda i: (i, 0)),
            pl.BlockSpec((BN, 16), lambda i: (i, 0)),
            pl.BlockSpec((1, n_cls), lambda i: (0, 0)),
            pl.BlockSpec((8, n_cls), lambda i: (0, 0)),
        ],
        out_specs=pl.BlockSpec((BN, n_cls), lambda i: (i, 0)),
        out_shape=jax.ShapeDtypeStruct((N, n_cls), f32),
    )(acc2, acc2, hx2, adst2, b2r, p8)

    return out


# speedup vs baseline: 1.3712x; 1.1389x over previous
import functools

import jax
import jax.numpy as jnp
from jax import lax
from jax.experimental import pallas as pl
from jax.experimental.pallas import tpu as pltpu
from jax.experimental.pallas import tpu_sc as plsc

NC, NS, L = 2, 16, 16
NW = NC * NS


def _dense1_body(x_ref, w1t_ref, asrc_map_ref, adst_map_ref, hx_ref, adst_ref):
    h = jnp.dot(x_ref[...], w1t_ref[...], preferred_element_type=jnp.float32)
    asrc = jnp.dot(h, asrc_map_ref[...], precision=lax.Precision.HIGHEST)
    zpad = jnp.zeros((h.shape[0], 8), jnp.float32)
    hx_ref[...] = jnp.concatenate([h, asrc, zpad], axis=1)
    adst_ref[...] = jnp.dot(h, adst_map_ref[...], precision=lax.Precision.HIGHEST)


def _combine1_body(acc0_ref, acc1_ref, hx_ref, adst_ref, b1_ref, w2t_ref,
                   a2s_map_ref, a2d_map_ref, bc8_ref, hx2_ref, adst2_ref):
    acc0 = acc0_ref[0]
    acc1 = acc1_ref[0]
    asrc = hx_ref[:, 128:136]
    ad = adst_ref[:, 0:8]
    a = asrc + ad
    a = jnp.where(a > 0, a, 0.2 * a)
    exs = jnp.exp(a)
    den = acc0[:, 128:136] + acc1[:, 128:136] + exs
    h = hx_ref[:, 0:128]
    bc8 = bc8_ref[...]
    exs_b = jnp.dot(exs, bc8, precision=lax.Precision.HIGHEST)
    num = acc0[:, 0:128] + acc1[:, 0:128] + exs_b * h
    recip = 1.0 / (den + 1e-16)
    recip_b = jnp.dot(recip, bc8, precision=lax.Precision.HIGHEST)
    out1 = num * recip_b + b1_ref[...]
    g = jnp.where(out1 > 0, out1, jnp.exp(out1) - 1.0)
    h2 = jnp.dot(g, w2t_ref[...], preferred_element_type=jnp.float32)
    asrc2 = jnp.dot(h2, a2s_map_ref[...], precision=lax.Precision.HIGHEST)
    zpad = jnp.zeros((h2.shape[0], 8), jnp.float32)
    hx2_ref[...] = jnp.concatenate([h2, asrc2, zpad], axis=1)
    adst2_ref[...] = jnp.dot(h2, a2d_map_ref[...], precision=lax.Precision.HIGHEST)


def _combine2_body(acc0_ref, acc1_ref, hx2_ref, adst2_ref, b2_ref, p8_ref,
                   out_ref):
    acc0 = acc0_ref[0]
    acc1 = acc1_ref[0]
    asrc = hx2_ref[:, 64:72]
    ad = adst2_ref[:, 0:8]
    a = asrc + ad
    a = jnp.where(a > 0, a, 0.2 * a)
    exs = jnp.exp(a)
    den = acc0[:, 64:72] + acc1[:, 64:72] + exs
    h2 = hx2_ref[:, 0:64]
    p8 = p8_ref[...]
    exs_b = jnp.dot(exs, p8, precision=lax.Precision.HIGHEST)
    num = acc0[:, 0:64] + acc1[:, 0:64] + exs_b * h2
    recip = 1.0 / (den + 1e-16)
    recip_b = jnp.dot(recip, p8, precision=lax.Precision.HIGHEST)
    out_ref[...] = num * recip_b + b2_ref[...]


def _make_edge_kernel(D, NH, ACC_N, CH, C):
    W = D + 16
    HB = D // NH
    RPT = ACC_N // NS
    mesh = plsc.VectorSubcoreMesh(core_axis_name="c", subcore_axis_name="s")

    assert CH % 4 == 0 and CH >= 8

    @functools.partial(
        pl.kernel,
        out_type=jax.ShapeDtypeStruct((NC, ACC_N, W), jnp.float32),
        mesh=mesh,
        compiler_params=pltpu.CompilerParams(use_tc_tiling_on_sc=False,
                                             needs_layout_passes=False),
        scratch_types=[
            [pltpu.VMEM((C,), jnp.int32)] * 4,
            [pltpu.VMEM((C,), jnp.int32)] * 4,
            [pltpu.VMEM((C,), jnp.int32)] * 4,
            [pltpu.VMEM((C, W), jnp.float32)] * 2,
            [pltpu.VMEM((C, 16), jnp.float32)] * 2,
            [pltpu.VMEM((C, W), jnp.float32)] * 2,
            pltpu.VMEM_SHARED((ACC_N, W), jnp.float32),
            [pltpu.SemaphoreType.DMA] * 2,
            [pltpu.SemaphoreType.DMA] * 2,
            [pltpu.SemaphoreType.DMA] * 4,
            [pltpu.SemaphoreType.DMA] * 4,
        ],
    )
    def edge_kernel(hx_hbm, adst_hbm, src_hbm, dstg_hbm, dsts_hbm, out_hbm,
                    srcv, dstgv, dstsv, hxv, adstv, msgv, acc, gsem, ssem,
                    isem, dsem):
        cid = lax.axis_index("c")
        sid = lax.axis_index("s")
        wid = sid * NC + cid

        def _zero_row(r, carry):
            for c0 in range(W // 16):
                msgv[0][r, pl.ds(c0 * 16, 16)] = jnp.zeros((16,), jnp.float32)
            return carry
        lax.fori_loop(0, C, _zero_row, 0)
        row0 = sid * RPT
        off = 0
        while off < RPT:
            nb = min(C, RPT - off)
            pltpu.sync_copy(msgv[0].at[pl.ds(0, nb)],
                            acc.at[pl.ds(row0 + off, nb)])
            off += nb
        plsc.subcore_barrier()

        ebase = wid * (CH * C)

        def _turn(b, s, ci):
            s2 = (s + 2) % 4

            pltpu.make_async_copy(hx_hbm.at[srcv[s]], hxv[b], gsem[b]).wait()
            pltpu.make_async_copy(adst_hbm.at[dstgv[s]], adstv[b],
                                  gsem[b]).wait()
            @pl.when(ci + 4 < CH)
            def _():
                base4 = ebase + (ci + 4) * C
                pltpu.async_copy(src_hbm.at[pl.ds(base4, C)], srcv[s],
                                 isem[s])
                pltpu.async_copy(dstg_hbm.at[pl.ds(base4, C)], dstgv[s],
                                 isem[s])

            @pl.when(ci >= 2)
            def _():
                pltpu.make_async_copy(msgv[b], acc.at[dstsv[s2]],
                                      ssem[b]).wait()
            @pl.when(ci + 2 < CH)
            def _():
                pltpu.async_copy(dsts_hbm.at[pl.ds(ebase + (ci + 2) * C, C)],
                                 dstsv[s2], dsem[s2])

            def _ex4(i, ecarry):
                for j in range(4):
                    e = i * 4 + j
                    asrc = hxv[b][e, pl.ds(D, 16)]
                    ad = adstv[b][e, :]
                    a = asrc + ad
                    a = jnp.where(a > 0, a, 0.2 * a)
                    msgv[b][e, pl.ds(D, 16)] = jnp.exp(a)
                return ecarry
            lax.fori_loop(0, C // 4, _ex4, 0)

            def _msg4(i, ecarry):
                for j in range(4):
                    e = i * 4 + j
                    exv = msgv[b][e, pl.ds(D, 16)]
                    for hd in range(NH):
                        bb = jnp.broadcast_to(exv[hd], (L,))
                        for v in range(HB // 16):
                            c0 = hd * HB + v * 16
                            msgv[b][e, pl.ds(c0, 16)] = (
                                hxv[b][e, pl.ds(c0, 16)] * bb)
                return ecarry
            lax.fori_loop(0, C // 4, _msg4, 0)

            @pl.when(ci >= 2)
            def _():
                pltpu.make_async_copy(dsts_hbm.at[pl.ds(ebase + ci * C, C)],
                                      dstsv[s], dsem[s]).wait()
            pltpu.async_copy(msgv[b], acc.at[dstsv[s]], ssem[b], add=True)

            @pl.when(ci + 2 < CH)
            def _():
                @pl.when(ci >= 2)
                def _():
                    base2 = ebase + (ci + 2) * C
                    pltpu.make_async_copy(src_hbm.at[pl.ds(base2, C)],
                                          srcv[s2], isem[s2]).wait()
                    pltpu.make_async_copy(dstg_hbm.at[pl.ds(base2, C)],
                                          dstgv[s2], isem[s2]).wait()
                pltpu.async_copy(hx_hbm.at[srcv[s2]], hxv[b], gsem[b])
                pltpu.async_copy(adst_hbm.at[dstgv[s2]], adstv[b], gsem[b])

        for s in range(4):
            pltpu.sync_copy(src_hbm.at[pl.ds(ebase + s * C, C)], srcv[s])
            pltpu.sync_copy(dstg_hbm.at[pl.ds(ebase + s * C, C)], dstgv[s])
        for s in range(2):
            pltpu.sync_copy(dsts_hbm.at[pl.ds(ebase + s * C, C)], dstsv[s])
            pltpu.async_copy(hx_hbm.at[srcv[s]], hxv[s], gsem[s])
            pltpu.async_copy(adst_hbm.at[dstgv[s]], adstv[s], gsem[s])

        def _quad(k, carry):
            ci = 4 * k
            _turn(0, 0, ci)
            _turn(1, 1, ci + 1)
            _turn(0, 2, ci + 2)
            _turn(1, 3, ci + 3)
            return carry
        lax.fori_loop(0, CH // 4, _quad, 0)
        for b, s in ((0, (CH - 2) % 4), (1, (CH - 1) % 4)):
            pltpu.make_async_copy(msgv[b], acc.at[dstsv[s]], ssem[b]).wait()
        plsc.subcore_barrier()

        off = 0
        while off < RPT:
            nb = min(C, RPT - off)
            pltpu.sync_copy(acc.at[pl.ds(row0 + off, nb)],
                            msgv[0].at[pl.ds(0, nb)])
            pltpu.sync_copy(msgv[0].at[pl.ds(0, nb)],
                            out_hbm.at[cid, pl.ds(row0 + off, nb)])
            off += nb

    return edge_kernel


def kernel(x, edge_index, W1, att_src1, att_dst1, b1, W2, att_src2, att_dst2,
           b2):
    N, d_in = x.shape
    E = edge_index.shape[1]
    heads, hf = att_src1.shape[1], att_src1.shape[2]
    D1 = heads * hf
    n_cls = W2.shape[0]
    ACC_N = 10112
    f32 = jnp.float32

    def _pad_edges(C):
        EPC = NW * C
        CH = -(-E // EPC)
        CH = -(-CH // 4) * 4
        pad = CH * EPC - E
        src_p = jnp.concatenate([edge_index[0], jnp.zeros((pad,), jnp.int32)])
        dstg_p = jnp.concatenate([edge_index[1], jnp.zeros((pad,), jnp.int32)])
        dsts_p = jnp.concatenate([edge_index[1],
                                  jnp.full((pad,), N, jnp.int32)])
        return CH, src_p, dstg_p, dsts_p

    C1, C2 = 64, 128
    CH1, src_p1, dstg_p1, dsts_p1 = _pad_edges(C1)
    CH2, src_p2, dstg_p2, dsts_p2 = _pad_edges(C2)

    att1s = att_src1.reshape(D1)
    att1d = att_dst1.reshape(D1)
    headsel = (jnp.arange(D1)[:, None] // hf ==
               jnp.arange(heads)[None, :]).astype(f32)
    asrc_map = headsel * att1s[:, None]
    adst_map = jnp.pad(headsel * att1d[:, None], ((0, 0), (0, 8)))
    bc8 = headsel.T
    a2s_map = jnp.pad(att_src2.reshape(n_cls, 1), ((0, 0), (0, 7)))
    a2d_map = jnp.pad(att_dst2.reshape(n_cls, 1), ((0, 0), (0, 15)))
    p8 = jnp.zeros((8, n_cls), f32).at[0, :].set(1.0)
    b1r = b1.reshape(1, D1)
    b2r = b2.reshape(1, n_cls)

    BN = 1000
    hx, adst16 = pl.pallas_call(
        _dense1_body,
        grid=(N // BN,),
        in_specs=[
            pl.BlockSpec((BN, d_in), lambda i: (i, 0)),
            pl.BlockSpec((d_in, D1), lambda i: (0, 0)),
            pl.BlockSpec((D1, heads), lambda i: (0, 0)),
            pl.BlockSpec((D1, 16), lambda i: (0, 0)),
        ],
        out_specs=[
            pl.BlockSpec((BN, D1 + 16), lambda i: (i, 0)),
            pl.BlockSpec((BN, 16), lambda i: (i, 0)),
        ],
        out_shape=[
            jax.ShapeDtypeStruct((N, D1 + 16), f32),
            jax.ShapeDtypeStruct((N, 16), f32),
        ],
    )(x, W1.T, asrc_map, adst_map)

    edge1 = _make_edge_kernel(D1, heads, ACC_N, CH1, C1)
    acc1 = edge1(hx, adst16, src_p1, dstg_p1, dsts_p1)

    hx2, adst2 = pl.pallas_call(
        _combine1_body,
        grid=(N // BN,),
        in_specs=[
            pl.BlockSpec((1, BN, D1 + 16), lambda i: (0, i, 0)),
            pl.BlockSpec((1, BN, D1 + 16), lambda i: (1, i, 0)),
            pl.BlockSpec((BN, D1 + 16), lambda i: (i, 0)),
            pl.BlockSpec((BN, 16), lambda i: (i, 0)),
            pl.BlockSpec((1, D1), lambda i: (0, 0)),
            pl.BlockSpec((D1, n_cls), lambda i: (0, 0)),
            pl.BlockSpec((n_cls, 8), lambda i: (0, 0)),
            pl.BlockSpec((n_cls, 16), lambda i: (0, 0)),
            pl.BlockSpec((8, D1), lambda i: (0, 0)),
        ],
        out_specs=[
            pl.BlockSpec((BN, n_cls + 16), lambda i: (i, 0)),
            pl.BlockSpec((BN, 16), lambda i: (i, 0)),
        ],
        out_shape=[
            jax.ShapeDtypeStruct((N, n_cls + 16), f32),
            jax.ShapeDtypeStruct((N, 16), f32),
        ],
    )(acc1, acc1, hx, adst16, b1r, W2.T, a2s_map, a2d_map, bc8)

    edge2 = _make_edge_kernel(n_cls, 1, ACC_N, CH2, C2)
    acc2 = edge2(hx2, adst2, src_p2, dstg_p2, dsts_p2)

    out = pl.pallas_call(
        _combine2_body,
        grid=(N // BN,),
        in_specs=[
            pl.BlockSpec((1, BN, n_cls + 16), lambda i: (0, i, 0)),
            pl.BlockSpec((1, BN, n_cls + 16), lambda i: (1, i, 0)),
            pl.BlockSpec((BN, n_cls + 16), lambda i: (i, 0)),
            pl.BlockSpec((BN, 16), lambda i: (i, 0)),
            pl.BlockSpec((1, n_cls), lambda i: (0, 0)),
            pl.BlockSpec((8, n_cls), lambda i: (0, 0)),
        ],
        out_specs=pl.BlockSpec((BN, n_cls), lambda i: (i, 0)),
        out_shape=jax.ShapeDtypeStruct((N, n_cls), f32),
    )(acc2, acc2, hx2, adst2, b2r, p8)

    return out


# speedup vs baseline: 1.4985x; 1.0928x over previous
import functools

import jax
import jax.numpy as jnp
from jax import lax
from jax.experimental import pallas as pl
from jax.experimental.pallas import tpu as pltpu
from jax.experimental.pallas import tpu_sc as plsc

NC, NS, L = 2, 16, 16
NW = NC * NS


def _dense1_body(x_ref, w1t_ref, asrc_map_ref, adst_map_ref, hx_ref, adst_ref):
    h = jnp.dot(x_ref[...], w1t_ref[...], preferred_element_type=jnp.float32)
    asrc = jnp.dot(h, asrc_map_ref[...], precision=lax.Precision.HIGHEST)
    zpad = jnp.zeros((h.shape[0], 8), jnp.float32)
    hx_ref[...] = jnp.concatenate([h, asrc, zpad], axis=1)
    adst_ref[...] = jnp.dot(h, adst_map_ref[...], precision=lax.Precision.HIGHEST)


def _combine1_body(acc0_ref, acc1_ref, hx_ref, adst_ref, b1_ref, w2t_ref,
                   a2s_map_ref, a2d_map_ref, bc8_ref, p2_ref, hx2_ref,
                   adst2_ref):
    acc0 = acc0_ref[0]
    acc1 = acc1_ref[0]
    asrc = hx_ref[:, 128:136]
    ad = adst_ref[:, 0:8]
    a = asrc + ad
    a = jnp.where(a > 0, a, 0.2 * a)
    exs = jnp.exp(a)
    den = acc0[:, 128:136] + acc1[:, 128:136] + exs
    h = hx_ref[:, 0:128]
    bc8 = bc8_ref[...]
    exs_b = jnp.dot(exs, bc8, precision=lax.Precision.HIGHEST)
    accf = jnp.dot(acc0[:, 0:128] + acc1[:, 0:128], p2_ref[...],
                   precision=lax.Precision.HIGHEST)
    num = accf + exs_b * h
    recip = 1.0 / (den + 1e-16)
    recip_b = jnp.dot(recip, bc8, precision=lax.Precision.HIGHEST)
    out1 = num * recip_b + b1_ref[...]
    g = jnp.where(out1 > 0, out1, jnp.exp(out1) - 1.0)
    h2 = jnp.dot(g, w2t_ref[...], preferred_element_type=jnp.float32)
    asrc2 = jnp.dot(h2, a2s_map_ref[...], precision=lax.Precision.HIGHEST)
    zpad = jnp.zeros((h2.shape[0], 8), jnp.float32)
    hx2_ref[...] = jnp.concatenate([h2, asrc2, zpad], axis=1)
    adst2_ref[...] = jnp.dot(h2, a2d_map_ref[...], precision=lax.Precision.HIGHEST)


def _combine2_body(acc0_ref, acc1_ref, hx2_ref, adst2_ref, b2_ref, p8_ref,
                   p2c_ref, out_ref):
    acc0 = acc0_ref[0]
    acc1 = acc1_ref[0]
    asrc = hx2_ref[:, 64:72]
    ad = adst2_ref[:, 0:8]
    a = asrc + ad
    a = jnp.where(a > 0, a, 0.2 * a)
    exs = jnp.exp(a)
    den = acc0[:, 64:72] + acc1[:, 64:72] + exs
    h2 = hx2_ref[:, 0:64]
    p8 = p8_ref[...]
    exs_b = jnp.dot(exs, p8, precision=lax.Precision.HIGHEST)
    accf = jnp.dot(acc0[:, 0:64] + acc1[:, 0:64], p2c_ref[...],
                   precision=lax.Precision.HIGHEST)
    num = accf + exs_b * h2
    recip = 1.0 / (den + 1e-16)
    recip_b = jnp.dot(recip, p8, precision=lax.Precision.HIGHEST)
    out_ref[...] = num * recip_b + b2_ref[...]


def _make_edge_kernel(D, NH, ACC_N, CH, C):
    W = D + 16
    WG = D // 2 + 16
    HB = D // NH
    RPT = ACC_N // NS
    mesh = plsc.VectorSubcoreMesh(core_axis_name="c", subcore_axis_name="s")

    assert CH % 4 == 0 and CH >= 8

    @functools.partial(
        pl.kernel,
        out_type=jax.ShapeDtypeStruct((NC, ACC_N, W), jnp.float32),
        mesh=mesh,
        compiler_params=pltpu.CompilerParams(use_tc_tiling_on_sc=False,
                                             needs_layout_passes=False),
        scratch_types=[
            [pltpu.VMEM((C,), jnp.int32)] * 4,
            [pltpu.VMEM((C,), jnp.int32)] * 4,
            [pltpu.VMEM((C,), jnp.int32)] * 4,
            [pltpu.VMEM((C, WG), jnp.float32)] * 2,
            [pltpu.VMEM((C, 16), jnp.float32)] * 2,
            [pltpu.VMEM((C, W), jnp.float32)] * 2,
            pltpu.VMEM_SHARED((ACC_N, W), jnp.float32),
            [pltpu.SemaphoreType.DMA] * 2,
            [pltpu.SemaphoreType.DMA] * 2,
            [pltpu.SemaphoreType.DMA] * 4,
            [pltpu.SemaphoreType.DMA] * 4,
        ],
    )
    def edge_kernel(hx_hbm, adst_hbm, src_hbm, dstg_hbm, dsts_hbm, out_hbm,
                    srcv, dstgv, dstsv, hxv, adstv, msgv, acc, gsem, ssem,
                    isem, dsem):
        cid = lax.axis_index("c")
        sid = lax.axis_index("s")
        wid = sid * NC + cid

        def _zero_row(r, carry):
            for c0 in range(W // 16):
                msgv[0][r, pl.ds(c0 * 16, 16)] = jnp.zeros((16,), jnp.float32)
            return carry
        lax.fori_loop(0, C, _zero_row, 0)
        row0 = sid * RPT
        off = 0
        while off < RPT:
            nb = min(C, RPT - off)
            pltpu.sync_copy(msgv[0].at[pl.ds(0, nb)],
                            acc.at[pl.ds(row0 + off, nb)])
            off += nb
        plsc.subcore_barrier()

        ebase = wid * (CH * C)

        def _turn(b, s, ci):
            s2 = (s + 2) % 4

            pltpu.make_async_copy(hx_hbm.at[srcv[s]], hxv[b], gsem[b]).wait()
            pltpu.make_async_copy(adst_hbm.at[dstgv[s]], adstv[b],
                                  gsem[b]).wait()
            @pl.when(ci + 4 < CH)
            def _():
                base4 = ebase + (ci + 4) * C
                pltpu.async_copy(src_hbm.at[pl.ds(base4, C)], srcv[s],
                                 isem[s])
                pltpu.async_copy(dstg_hbm.at[pl.ds(base4, C)], dstgv[s],
                                 isem[s])

            @pl.when(ci >= 2)
            def _():
                pltpu.make_async_copy(msgv[b], acc.at[dstsv[s2]],
                                      ssem[b]).wait()
            @pl.when(ci + 2 < CH)
            def _():
                pltpu.async_copy(dsts_hbm.at[pl.ds(ebase + (ci + 2) * C, C)],
                                 dstsv[s2], dsem[s2])

            def _ex4(i, ecarry):
                for j in range(4):
                    e = i * 4 + j
                    asrc = hxv[b][e, pl.ds(D // 2, 16)]
                    ad = adstv[b][e, :]
                    a = asrc + ad
                    a = jnp.where(a > 0, a, 0.2 * a)
                    ex = jnp.exp(a)
                    adstv[b][e, :] = ex
                    msgv[b][e, pl.ds(D, 16)] = ex
                return ecarry
            lax.fori_loop(0, C // 4, _ex4, 0)

            half = lax.iota(jnp.int32, L) // (HB // 2)
            cidxs = [jnp.full((L,), 32 * blk // HB, jnp.int32) + half
                     for blk in range(D // 32)]
            himask = jnp.full((L,), -65536, jnp.int32)
            def _msg4(i, ecarry):
                for j in range(4):
                    e = i * 4 + j
                    eidx = jnp.broadcast_to(e, (L,)).astype(jnp.int32)
                    for blk in range(D // 32):
                        vw = hxv[b][e, pl.ds(16 * blk, 16)]
                        vi = lax.bitcast_convert_type(vw, jnp.int32)
                        ea = lax.bitcast_convert_type(
                            lax.shift_left(vi, 16), jnp.float32)
                        eo = lax.bitcast_convert_type(
                            jnp.bitwise_and(vi, himask), jnp.float32)
                        bb = plsc.load_gather(adstv[b], [eidx, cidxs[blk]])
                        msgv[b][e, pl.ds(32 * blk, 16)] = ea * bb
                        msgv[b][e, pl.ds(32 * blk + 16, 16)] = eo * bb
                return ecarry
            lax.fori_loop(0, C // 4, _msg4, 0)

            @pl.when(ci >= 2)
            def _():
                pltpu.make_async_copy(dsts_hbm.at[pl.ds(ebase + ci * C, C)],
                                      dstsv[s], dsem[s]).wait()
            pltpu.async_copy(msgv[b], acc.at[dstsv[s]], ssem[b], add=True)

            @pl.when(ci + 2 < CH)
            def _():
                @pl.when(ci >= 2)
                def _():
                    base2 = ebase + (ci + 2) * C
                    pltpu.make_async_copy(src_hbm.at[pl.ds(base2, C)],
                                          srcv[s2], isem[s2]).wait()
                    pltpu.make_async_copy(dstg_hbm.at[pl.ds(base2, C)],
                                          dstgv[s2], isem[s2]).wait()
                pltpu.async_copy(hx_hbm.at[srcv[s2]], hxv[b], gsem[b])
                pltpu.async_copy(adst_hbm.at[dstgv[s2]], adstv[b], gsem[b])

        for s in range(4):
            pltpu.sync_copy(src_hbm.at[pl.ds(ebase + s * C, C)], srcv[s])
            pltpu.sync_copy(dstg_hbm.at[pl.ds(ebase + s * C, C)], dstgv[s])
        for s in range(2):
            pltpu.sync_copy(dsts_hbm.at[pl.ds(ebase + s * C, C)], dstsv[s])
            pltpu.async_copy(hx_hbm.at[srcv[s]], hxv[s], gsem[s])
            pltpu.async_copy(adst_hbm.at[dstgv[s]], adstv[s], gsem[s])

        def _quad(k, carry):
            ci = 4 * k
            _turn(0, 0, ci)
            _turn(1, 1, ci + 1)
            _turn(0, 2, ci + 2)
            _turn(1, 3, ci + 3)
            return carry
        lax.fori_loop(0, CH // 4, _quad, 0)
        for b, s in ((0, (CH - 2) % 4), (1, (CH - 1) % 4)):
            pltpu.make_async_copy(msgv[b], acc.at[dstsv[s]], ssem[b]).wait()
        plsc.subcore_barrier()

        off = 0
        while off < RPT:
            nb = min(C, RPT - off)
            pltpu.sync_copy(acc.at[pl.ds(row0 + off, nb)],
                            msgv[0].at[pl.ds(0, nb)])
            pltpu.sync_copy(msgv[0].at[pl.ds(0, nb)],
                            out_hbm.at[cid, pl.ds(row0 + off, nb)])
            off += nb

    return edge_kernel


def kernel(x, edge_index, W1, att_src1, att_dst1, b1, W2, att_src2, att_dst2,
           b2):
    N, d_in = x.shape
    E = edge_index.shape[1]
    heads, hf = att_src1.shape[1], att_src1.shape[2]
    D1 = heads * hf
    n_cls = W2.shape[0]
    ACC_N = 10112
    f32 = jnp.float32

    def _pad_edges(C):
        EPC = NW * C
        CH = -(-E // EPC)
        CH = -(-CH // 4) * 4
        pad = CH * EPC - E
        src_p = jnp.concatenate([edge_index[0], jnp.zeros((pad,), jnp.int32)])
        dstg_p = jnp.concatenate([edge_index[1], jnp.zeros((pad,), jnp.int32)])
        dsts_p = jnp.concatenate([edge_index[1],
                                  jnp.full((pad,), N, jnp.int32)])
        return CH, src_p, dstg_p, dsts_p

    C1, C2 = 64, 128
    CH1, src_p1, dstg_p1, dsts_p1 = _pad_edges(C1)
    CH2, src_p2, dstg_p2, dsts_p2 = _pad_edges(C2)

    att1s = att_src1.reshape(D1)
    att1d = att_dst1.reshape(D1)
    headsel = (jnp.arange(D1)[:, None] // hf ==
               jnp.arange(heads)[None, :]).astype(f32)
    asrc_map = headsel * att1s[:, None]
    adst_map = jnp.pad(headsel * att1d[:, None], ((0, 0), (0, 8)))
    bc8 = headsel.T
    a2s_map = jnp.pad(att_src2.reshape(n_cls, 1), ((0, 0), (0, 7)))
    a2d_map = jnp.pad(att_dst2.reshape(n_cls, 1), ((0, 0), (0, 15)))
    p8 = jnp.zeros((8, n_cls), f32).at[0, :].set(1.0)
    b1r = b1.reshape(1, D1)
    b2r = b2.reshape(1, n_cls)

    def _pack_pairs(h):
        hb = h.astype(jnp.bfloat16).reshape(h.shape[0], -1, 2)
        return lax.bitcast_convert_type(hb, f32)

    def _unperm_mat(D):
        c = jnp.arange(D)
        blk, i = c // 32, c % 32
        orig = 32 * blk + 2 * (i % 16) + i // 16
        return (orig[:, None] == jnp.arange(D)[None, :]).astype(f32)

    p2_1 = _unperm_mat(D1)
    p2_2 = _unperm_mat(n_cls)

    BN = 1000
    hx, adst16 = pl.pallas_call(
        _dense1_body,
        grid=(N // BN,),
        in_specs=[
            pl.BlockSpec((BN, d_in), lambda i: (i, 0)),
            pl.BlockSpec((d_in, D1), lambda i: (0, 0)),
            pl.BlockSpec((D1, heads), lambda i: (0, 0)),
            pl.BlockSpec((D1, 16), lambda i: (0, 0)),
        ],
        out_specs=[
            pl.BlockSpec((BN, D1 + 16), lambda i: (i, 0)),
            pl.BlockSpec((BN, 16), lambda i: (i, 0)),
        ],
        out_shape=[
            jax.ShapeDtypeStruct((N, D1 + 16), f32),
            jax.ShapeDtypeStruct((N, 16), f32),
        ],
    )(x, W1.T, asrc_map, adst_map)

    hxp1 = jnp.concatenate([_pack_pairs(hx[:, 0:D1]), hx[:, D1:D1 + 16]], 1)
    edge1 = _make_edge_kernel(D1, heads, ACC_N, CH1, C1)
    acc1 = edge1(hxp1, adst16, src_p1, dstg_p1, dsts_p1)

    hx2, adst2 = pl.pallas_call(
        _combine1_body,
        grid=(N // BN,),
        in_specs=[
            pl.BlockSpec((1, BN, D1 + 16), lambda i: (0, i, 0)),
            pl.BlockSpec((1, BN, D1 + 16), lambda i: (1, i, 0)),
            pl.BlockSpec((BN, D1 + 16), lambda i: (i, 0)),
            pl.BlockSpec((BN, 16), lambda i: (i, 0)),
            pl.BlockSpec((1, D1), lambda i: (0, 0)),
            pl.BlockSpec((D1, n_cls), lambda i: (0, 0)),
            pl.BlockSpec((n_cls, 8), lambda i: (0, 0)),
            pl.BlockSpec((n_cls, 16), lambda i: (0, 0)),
            pl.BlockSpec((8, D1), lambda i: (0, 0)),
            pl.BlockSpec((D1, D1), lambda i: (0, 0)),
        ],
        out_specs=[
            pl.BlockSpec((BN, n_cls + 16), lambda i: (i, 0)),
            pl.BlockSpec((BN, 16), lambda i: (i, 0)),
        ],
        out_shape=[
            jax.ShapeDtypeStruct((N, n_cls + 16), f32),
            jax.ShapeDtypeStruct((N, 16), f32),
        ],
    )(acc1, acc1, hx, adst16, b1r, W2.T, a2s_map, a2d_map, bc8, p2_1)

    hxp2 = jnp.concatenate(
        [_pack_pairs(hx2[:, 0:n_cls]), hx2[:, n_cls:n_cls + 16]], 1)
    edge2 = _make_edge_kernel(n_cls, 1, ACC_N, CH2, C2)
    acc2 = edge2(hxp2, adst2, src_p2, dstg_p2, dsts_p2)

    out = pl.pallas_call(
        _combine2_body,
        grid=(N // BN,),
        in_specs=[
            pl.BlockSpec((1, BN, n_cls + 16), lambda i: (0, i, 0)),
            pl.BlockSpec((1, BN, n_cls + 16), lambda i: (1, i, 0)),
            pl.BlockSpec((BN, n_cls + 16), lambda i: (i, 0)),
            pl.BlockSpec((BN, 16), lambda i: (i, 0)),
            pl.BlockSpec((1, n_cls), lambda i: (0, 0)),
            pl.BlockSpec((8, n_cls), lambda i: (0, 0)),
            pl.BlockSpec((n_cls, n_cls), lambda i: (0, 0)),
        ],
        out_specs=pl.BlockSpec((BN, n_cls), lambda i: (i, 0)),
        out_shape=jax.ShapeDtypeStruct((N, n_cls), f32),
    )(acc2, acc2, hx2, adst2, b2r, p8, p2_2)

    return out


# speedup vs baseline: 1.5064x; 1.0052x over previous
import functools

import jax
import jax.numpy as jnp
from jax import lax
from jax.experimental import pallas as pl
from jax.experimental.pallas import tpu as pltpu
from jax.experimental.pallas import tpu_sc as plsc

NC, NS, L = 2, 16, 16
NW = NC * NS


def _dense1_body(x_ref, w1t_ref, asrc_map_ref, adst_map_ref, hx_ref, adst_ref):
    h = jnp.dot(x_ref[...], w1t_ref[...], preferred_element_type=jnp.float32)
    asrc = jnp.dot(h, asrc_map_ref[...], precision=lax.Precision.HIGHEST)
    zpad = jnp.zeros((h.shape[0], 8), jnp.float32)
    hx_ref[...] = jnp.concatenate([h, asrc, zpad], axis=1)
    adst_ref[...] = jnp.dot(h, adst_map_ref[...], precision=lax.Precision.HIGHEST)


def _combine1_body(acc0_ref, acc1_ref, hx_ref, adst_ref, b1_ref, w2t_ref,
                   a2s_map_ref, a2d_map_ref, bc8_ref, p2_ref, hx2_ref,
                   adst2_ref):
    acc0 = acc0_ref[0]
    acc1 = acc1_ref[0]
    asrc = hx_ref[:, 128:136]
    ad = adst_ref[:, 0:8]
    a = asrc + ad
    a = jnp.where(a > 0, a, 0.2 * a)
    exs = jnp.exp(a)
    den = acc0[:, 128:136] + acc1[:, 128:136] + exs
    h = hx_ref[:, 0:128]
    bc8 = bc8_ref[...]
    exs_b = jnp.dot(exs, bc8, precision=lax.Precision.HIGHEST)
    accf = jnp.dot(acc0[:, 0:128] + acc1[:, 0:128], p2_ref[...],
                   precision=lax.Precision.HIGHEST)
    num = accf + exs_b * h
    recip = 1.0 / (den + 1e-16)
    recip_b = jnp.dot(recip, bc8, precision=lax.Precision.HIGHEST)
    out1 = num * recip_b + b1_ref[...]
    g = jnp.where(out1 > 0, out1, jnp.exp(out1) - 1.0)
    h2 = jnp.dot(g, w2t_ref[...], preferred_element_type=jnp.float32)
    asrc2 = jnp.dot(h2, a2s_map_ref[...], precision=lax.Precision.HIGHEST)
    zpad = jnp.zeros((h2.shape[0], 8), jnp.float32)
    hx2_ref[...] = jnp.concatenate([h2, asrc2, zpad], axis=1)
    adst2_ref[...] = jnp.dot(h2, a2d_map_ref[...], precision=lax.Precision.HIGHEST)


def _combine2_body(acc0_ref, acc1_ref, hx2_ref, adst2_ref, b2_ref, p8_ref,
                   p2c_ref, out_ref):
    acc0 = acc0_ref[0]
    acc1 = acc1_ref[0]
    asrc = hx2_ref[:, 64:72]
    ad = adst2_ref[:, 0:8]
    a = asrc + ad
    a = jnp.where(a > 0, a, 0.2 * a)
    exs = jnp.exp(a)
    den = acc0[:, 64:72] + acc1[:, 64:72] + exs
    h2 = hx2_ref[:, 0:64]
    p8 = p8_ref[...]
    exs_b = jnp.dot(exs, p8, precision=lax.Precision.HIGHEST)
    accf = jnp.dot(acc0[:, 0:64] + acc1[:, 0:64], p2c_ref[...],
                   precision=lax.Precision.HIGHEST)
    num = accf + exs_b * h2
    recip = 1.0 / (den + 1e-16)
    recip_b = jnp.dot(recip, p8, precision=lax.Precision.HIGHEST)
    out_ref[...] = num * recip_b + b2_ref[...]


def _make_edge_kernel(D, NH, ACC_N, CH, C):
    W = D + 16
    WG = D // 2 + 16
    HB = D // NH
    RPT = ACC_N // NS
    mesh = plsc.VectorSubcoreMesh(core_axis_name="c", subcore_axis_name="s")

    assert CH % 4 == 0 and CH >= 8

    @functools.partial(
        pl.kernel,
        out_type=jax.ShapeDtypeStruct((NC, ACC_N, W), jnp.float32),
        mesh=mesh,
        compiler_params=pltpu.CompilerParams(use_tc_tiling_on_sc=False,
                                             needs_layout_passes=False),
        scratch_types=[
            [pltpu.VMEM((C,), jnp.int32)] * 4,
            [pltpu.VMEM((C,), jnp.int32)] * 4,
            [pltpu.VMEM((C,), jnp.int32)] * 4,
            [pltpu.VMEM((C, WG), jnp.float32)] * 2,
            [pltpu.VMEM((C, 16), jnp.float32)] * 2,
            [pltpu.VMEM((C, W), jnp.float32)] * 2,
            pltpu.VMEM_SHARED((ACC_N, W), jnp.float32),
            [pltpu.SemaphoreType.DMA] * 2,
            [pltpu.SemaphoreType.DMA] * 2,
            [pltpu.SemaphoreType.DMA] * 4,
            [pltpu.SemaphoreType.DMA] * 4,
        ],
    )
    def edge_kernel(hx_hbm, adst_hbm, src_hbm, dstg_hbm, dsts_hbm, out_hbm,
                    srcv, dstgv, dstsv, hxv, adstv, msgv, acc, gsem, ssem,
                    isem, dsem):
        cid = lax.axis_index("c")
        sid = lax.axis_index("s")
        wid = sid * NC + cid

        def _zero_row(r, carry):
            for c0 in range(W // 16):
                msgv[0][r, pl.ds(c0 * 16, 16)] = jnp.zeros((16,), jnp.float32)
            return carry
        lax.fori_loop(0, C, _zero_row, 0)
        row0 = sid * RPT
        off = 0
        while off < RPT:
            nb = min(C, RPT - off)
            pltpu.sync_copy(msgv[0].at[pl.ds(0, nb)],
                            acc.at[pl.ds(row0 + off, nb)])
            off += nb
        plsc.subcore_barrier()

        ebase = wid * (CH * C)

        def _turn(b, s, ci):
            s2 = (s + 2) % 4

            pltpu.make_async_copy(hx_hbm.at[srcv[s]], hxv[b], gsem[b]).wait()
            pltpu.make_async_copy(adst_hbm.at[dstgv[s]], adstv[b],
                                  gsem[b]).wait()
            @pl.when(ci + 4 < CH)
            def _():
                base4 = ebase + (ci + 4) * C
                pltpu.async_copy(src_hbm.at[pl.ds(base4, C)], srcv[s],
                                 isem[s])
                pltpu.async_copy(dstg_hbm.at[pl.ds(base4, C)], dstgv[s],
                                 isem[s])

            @pl.when(ci >= 2)
            def _():
                pltpu.make_async_copy(msgv[b], acc.at[dstsv[s2]],
                                      ssem[b]).wait()
            @pl.when(ci + 2 < CH)
            def _():
                pltpu.async_copy(dsts_hbm.at[pl.ds(ebase + (ci + 2) * C, C)],
                                 dstsv[s2], dsem[s2])

            def _ex4(i, ecarry):
                for j in range(4):
                    e = i * 4 + j
                    asrc = hxv[b][e, pl.ds(D // 2, 16)]
                    ad = adstv[b][e, :]
                    a = asrc + ad
                    a = jnp.where(a > 0, a, 0.2 * a)
                    ex = jnp.exp(a)
                    adstv[b][e, :] = ex
                    msgv[b][e, pl.ds(D, 16)] = ex
                return ecarry
            lax.fori_loop(0, C // 4, _ex4, 0)

            half = lax.iota(jnp.int32, L) // (HB // 2)
            cidxs = [jnp.full((L,), 32 * blk // HB, jnp.int32) + half
                     for blk in range(D // 32)]
            himask = jnp.full((L,), -65536, jnp.int32)
            def _msg4(i, ecarry):
                for j in range(4):
                    e = i * 4 + j
                    eidx = jnp.broadcast_to(e, (L,)).astype(jnp.int32)
                    for blk in range(D // 32):
                        vw = hxv[b][e, pl.ds(16 * blk, 16)]
                        vi = lax.bitcast_convert_type(vw, jnp.int32)
                        ea = lax.bitcast_convert_type(
                            lax.shift_left(vi, 16), jnp.float32)
                        eo = lax.bitcast_convert_type(
                            jnp.bitwise_and(vi, himask), jnp.float32)
                        bb = plsc.load_gather(adstv[b], [eidx, cidxs[blk]])
                        msgv[b][e, pl.ds(32 * blk, 16)] = ea * bb
                        msgv[b][e, pl.ds(32 * blk + 16, 16)] = eo * bb
                return ecarry
            lax.fori_loop(0, C // 4, _msg4, 0)

            @pl.when(ci >= 2)
            def _():
                pltpu.make_async_copy(dsts_hbm.at[pl.ds(ebase + ci * C, C)],
                                      dstsv[s], dsem[s]).wait()
            pltpu.async_copy(msgv[b], acc.at[dstsv[s]], ssem[b], add=True)

            @pl.when(ci + 2 < CH)
            def _():
                @pl.when(ci >= 2)
                def _():
                    base2 = ebase + (ci + 2) * C
                    pltpu.make_async_copy(src_hbm.at[pl.ds(base2, C)],
                                          srcv[s2], isem[s2]).wait()
                    pltpu.make_async_copy(dstg_hbm.at[pl.ds(base2, C)],
                                          dstgv[s2], isem[s2]).wait()
                pltpu.async_copy(hx_hbm.at[srcv[s2]], hxv[b], gsem[b])
                pltpu.async_copy(adst_hbm.at[dstgv[s2]], adstv[b], gsem[b])

        for s in range(4):
            pltpu.sync_copy(src_hbm.at[pl.ds(ebase + s * C, C)], srcv[s])
            pltpu.sync_copy(dstg_hbm.at[pl.ds(ebase + s * C, C)], dstgv[s])
        for s in range(2):
            pltpu.sync_copy(dsts_hbm.at[pl.ds(ebase + s * C, C)], dstsv[s])
            pltpu.async_copy(hx_hbm.at[srcv[s]], hxv[s], gsem[s])
            pltpu.async_copy(adst_hbm.at[dstgv[s]], adstv[s], gsem[s])

        def _quad(k, carry):
            ci = 4 * k
            _turn(0, 0, ci)
            _turn(1, 1, ci + 1)
            _turn(0, 2, ci + 2)
            _turn(1, 3, ci + 3)
            return carry
        lax.fori_loop(0, CH // 4, _quad, 0)
        for b, s in ((0, (CH - 2) % 4), (1, (CH - 1) % 4)):
            pltpu.make_async_copy(msgv[b], acc.at[dstsv[s]], ssem[b]).wait()
        plsc.subcore_barrier()

        off = 0
        while off < RPT:
            nb = min(C, RPT - off)
            pltpu.sync_copy(acc.at[pl.ds(row0 + off, nb)],
                            msgv[0].at[pl.ds(0, nb)])
            pltpu.sync_copy(msgv[0].at[pl.ds(0, nb)],
                            out_hbm.at[cid, pl.ds(row0 + off, nb)])
            off += nb

    return edge_kernel


def kernel(x, edge_index, W1, att_src1, att_dst1, b1, W2, att_src2, att_dst2,
           b2):
    N, d_in = x.shape
    E = edge_index.shape[1]
    heads, hf = att_src1.shape[1], att_src1.shape[2]
    D1 = heads * hf
    n_cls = W2.shape[0]
    ACC_N = 10112
    f32 = jnp.float32

    def _pad_edges(C):
        EPC = NW * C
        CH = -(-E // EPC)
        CH = -(-CH // 4) * 4
        pad = CH * EPC - E
        src_p = jnp.concatenate([edge_index[0], jnp.zeros((pad,), jnp.int32)])
        dstg_p = jnp.concatenate([edge_index[1], jnp.zeros((pad,), jnp.int32)])
        dsts_p = jnp.concatenate([edge_index[1],
                                  jnp.full((pad,), N, jnp.int32)])
        return CH, src_p, dstg_p, dsts_p

    C1, C2 = 80, 128
    CH1, src_p1, dstg_p1, dsts_p1 = _pad_edges(C1)
    CH2, src_p2, dstg_p2, dsts_p2 = _pad_edges(C2)

    att1s = att_src1.reshape(D1)
    att1d = att_dst1.reshape(D1)
    headsel = (jnp.arange(D1)[:, None] // hf ==
               jnp.arange(heads)[None, :]).astype(f32)
    asrc_map = headsel * att1s[:, None]
    adst_map = jnp.pad(headsel * att1d[:, None], ((0, 0), (0, 8)))
    bc8 = headsel.T
    a2s_map = jnp.pad(att_src2.reshape(n_cls, 1), ((0, 0), (0, 7)))
    a2d_map = jnp.pad(att_dst2.reshape(n_cls, 1), ((0, 0), (0, 15)))
    p8 = jnp.zeros((8, n_cls), f32).at[0, :].set(1.0)
    b1r = b1.reshape(1, D1)
    b2r = b2.reshape(1, n_cls)

    def _pack_pairs(h):
        hb = h.astype(jnp.bfloat16).reshape(h.shape[0], -1, 2)
        return lax.bitcast_convert_type(hb, f32)

    def _unperm_mat(D):
        c = jnp.arange(D)
        blk, i = c // 32, c % 32
        orig = 32 * blk + 2 * (i % 16) + i // 16
        return (orig[:, None] == jnp.arange(D)[None, :]).astype(f32)

    p2_1 = _unperm_mat(D1)
    p2_2 = _unperm_mat(n_cls)

    BN = 1000
    hx, adst16 = pl.pallas_call(
        _dense1_body,
        grid=(N // BN,),
        in_specs=[
            pl.BlockSpec((BN, d_in), lambda i: (i, 0)),
            pl.BlockSpec((d_in, D1), lambda i: (0, 0)),
            pl.BlockSpec((D1, heads), lambda i: (0, 0)),
            pl.BlockSpec((D1, 16), lambda i: (0, 0)),
        ],
        out_specs=[
            pl.BlockSpec((BN, D1 + 16), lambda i: (i, 0)),
            pl.BlockSpec((BN, 16), lambda i: (i, 0)),
        ],
        out_shape=[
            jax.ShapeDtypeStruct((N, D1 + 16), f32),
            jax.ShapeDtypeStruct((N, 16), f32),
        ],
    )(x, W1.T, asrc_map, adst_map)

    hxp1 = jnp.concatenate([_pack_pairs(hx[:, 0:D1]), hx[:, D1:D1 + 16]], 1)
    edge1 = _make_edge_kernel(D1, heads, ACC_N, CH1, C1)
    acc1 = edge1(hxp1, adst16, src_p1, dstg_p1, dsts_p1)

    hx2, adst2 = pl.pallas_call(
        _combine1_body,
        grid=(N // BN,),
        in_specs=[
            pl.BlockSpec((1, BN, D1 + 16), lambda i: (0, i, 0)),
            pl.BlockSpec((1, BN, D1 + 16), lambda i: (1, i, 0)),
            pl.BlockSpec((BN, D1 + 16), lambda i: (i, 0)),
            pl.BlockSpec((BN, 16), lambda i: (i, 0)),
            pl.BlockSpec((1, D1), lambda i: (0, 0)),
            pl.BlockSpec((D1, n_cls), lambda i: (0, 0)),
            pl.BlockSpec((n_cls, 8), lambda i: (0, 0)),
            pl.BlockSpec((n_cls, 16), lambda i: (0, 0)),
            pl.BlockSpec((8, D1), lambda i: (0, 0)),
            pl.BlockSpec((D1, D1), lambda i: (0, 0)),
        ],
        out_specs=[
            pl.BlockSpec((BN, n_cls + 16), lambda i: (i, 0)),
            pl.BlockSpec((BN, 16), lambda i: (i, 0)),
        ],
        out_shape=[
            jax.ShapeDtypeStruct((N, n_cls + 16), f32),
            jax.ShapeDtypeStruct((N, 16), f32),
        ],
    )(acc1, acc1, hx, adst16, b1r, W2.T, a2s_map, a2d_map, bc8, p2_1)

    hxp2 = jnp.concatenate(
        [_pack_pairs(hx2[:, 0:n_cls]), hx2[:, n_cls:n_cls + 16]], 1)
    edge2 = _make_edge_kernel(n_cls, 1, ACC_N, CH2, C2)
    acc2 = edge2(hxp2, adst2, src_p2, dstg_p2, dsts_p2)

    out = pl.pallas_call(
        _combine2_body,
        grid=(N // BN,),
        in_specs=[
            pl.BlockSpec((1, BN, n_cls + 16), lambda i: (0, i, 0)),
            pl.BlockSpec((1, BN, n_cls + 16), lambda i: (1, i, 0)),
            pl.BlockSpec((BN, n_cls + 16), lambda i: (i, 0)),
            pl.BlockSpec((BN, 16), lambda i: (i, 0)),
            pl.BlockSpec((1, n_cls), lambda i: (0, 0)),
            pl.BlockSpec((8, n_cls), lambda i: (0, 0)),
            pl.BlockSpec((n_cls, n_cls), lambda i: (0, 0)),
        ],
        out_specs=pl.BlockSpec((BN, n_cls), lambda i: (i, 0)),
        out_shape=jax.ShapeDtypeStruct((N, n_cls), f32),
    )(acc2, acc2, hx2, adst2, b2r, p8, p2_2)

    return out
